# baseline (speedup 1.0000x reference)
import functools

import jax
import jax.numpy as jnp
from jax import lax
from jax.experimental import pallas as pl
from jax.experimental.pallas import tpu as pltpu
from jax.experimental.pallas import tpu_sc as plsc

LANES = 16
EPS = 1e-6


def _rsqrt_scalar(v):
    i = lax.bitcast_convert_type(v, jnp.int32)
    i = jnp.int32(0x5F3759DF) - lax.shift_right_logical(i, 1)
    y = lax.bitcast_convert_type(i, jnp.float32)
    for _ in range(3):
        y = y * (1.5 - 0.5 * v * y * y)
    return y


@functools.lru_cache(maxsize=None)
def _build(ntok, hidden, tvocab, chunk):
    info = plsc.get_sparse_core_info()
    nw = info.num_cores * info.num_subcores
    assert ntok % (nw * chunk) == 0
    assert tvocab == 2
    tpw = ntok // nw
    nchunks = tpw // chunk
    dchunks = hidden // LANES
    inv_h = 1.0 / hidden
    mesh = plsc.VectorSubcoreMesh(core_axis_name="c", subcore_axis_name="s")

    @functools.partial(
        pl.kernel,
        out_type=jax.ShapeDtypeStruct((ntok, hidden), jnp.float32),
        mesh=mesh,
        scratch_types=[
            pltpu.VMEM((chunk,), jnp.int32),
            pltpu.VMEM((chunk,), jnp.int32),
            pltpu.VMEM((chunk + LANES,), jnp.int32),
            pltpu.VMEM((chunk, hidden), jnp.float32),
            pltpu.VMEM((chunk, hidden), jnp.float32),
            pltpu.VMEM((tvocab, hidden), jnp.float32),
            pltpu.VMEM((hidden,), jnp.float32),
            pltpu.VMEM((hidden,), jnp.float32),
            pltpu.VMEM((2 * LANES,), jnp.float32),
            pltpu.SemaphoreType.DMA,
            pltpu.SemaphoreType.DMA,
        ],
    )
    def sc_kernel(ids_hbm, pids_hbm, tids_hbm, wtab_hbm, ptab_hbm, ttab_hbm,
                  lnw_hbm, lnb_hbm, out_hbm,
                  widx_v, pidx_v, tids_v, wrows, prows, ttab_v, lnw_v, lnb_v,
                  red_v, sem_w, sem_p):
        wid = lax.axis_index("s") * info.num_cores + lax.axis_index("c")
        base = wid * tpw

        red_v[pl.ds(LANES, LANES)] = jnp.zeros((LANES,), jnp.float32)

        def lane_sum(v):
            for sh in (8, 4, 2, 1):
                red_v[pl.ds(0, LANES)] = v
                v = v + red_v[pl.ds(sh, LANES)]
            return v[0]

        pltpu.sync_copy(ttab_hbm, ttab_v)
        pltpu.sync_copy(lnw_hbm, lnw_v)
        pltpu.sync_copy(lnb_hbm, lnb_v)
        for d in range(dchunks):
            sl = pl.ds(d * LANES, LANES)
            ttab_v[1, sl] = ttab_v[1, sl] - ttab_v[0, sl]

        def chunk_body(c, carry):
            cbase = base + c * chunk
            pltpu.sync_copy(ids_hbm.at[pl.ds(cbase, chunk)], widx_v)
            pltpu.sync_copy(pids_hbm.at[pl.ds(cbase, chunk)], pidx_v)
            pltpu.sync_copy(tids_hbm.at[pl.ds(cbase, chunk)],
                            tids_v.at[pl.ds(0, chunk)])
            cp_w = pltpu.async_copy(wtab_hbm.at[widx_v], wrows, sem_w)
            cp_p = pltpu.async_copy(ptab_hbm.at[pidx_v], prows, sem_p)
            cp_w.wait()
            cp_p.wait()

            def tok_body(i, tcarry):
                tf = tids_v[pl.ds(i, LANES)][0].astype(jnp.float32)
                acc1 = jnp.zeros((LANES,), jnp.float32)
                acc2 = jnp.zeros((LANES,), jnp.float32)
                for d in range(dchunks):
                    sl = pl.ds(d * LANES, LANES)
                    t = ttab_v[0, sl] + tf * ttab_v[1, sl]
                    x = wrows[i, sl] + prows[i, sl] + t
                    wrows[i, sl] = x
                    acc1 = acc1 + x
                    acc2 = acc2 + x * x
                mean = lane_sum(acc1) * inv_h
                var = lane_sum(acc2) * inv_h - mean * mean
                r = _rsqrt_scalar(var + EPS)
                for d in range(dchunks):
                    sl = pl.ds(d * LANES, LANES)
                    y = (wrows[i, sl] - mean) * (r * lnw_v[sl]) + lnb_v[sl]
                    wrows[i, sl] = y
                return tcarry

            lax.fori_loop(0, chunk, tok_body, 0)
            pltpu.sync_copy(wrows, out_hbm.at[pl.ds(cbase, chunk)])
            return carry

        lax.fori_loop(0, nchunks, chunk_body, 0)

    return sc_kernel


def kernel(input_ids, token_type_ids, position_ids, attention_mask,
           word_emb, pos_emb, type_emb, ln_weight, ln_bias):
    b, s = input_ids.shape
    ntok = b * s
    hidden = word_emb.shape[1]
    ids = input_ids.reshape(ntok).astype(jnp.int32)
    pids = position_ids.reshape(ntok).astype(jnp.int32)
    tids = token_type_ids.reshape(ntok).astype(jnp.int32)
    fn = _build(ntok, hidden, type_emb.shape[0], 32)
    out = fn(ids, pids, tids, word_emb, pos_emb, type_emb,
             ln_weight, ln_bias)
    return out.reshape(b, s, hidden)


# speedup vs baseline: 1.0513x; 1.0513x over previous
import functools

import jax
import jax.numpy as jnp
from jax import lax
from jax.experimental import pallas as pl
from jax.experimental.pallas import tpu as pltpu
from jax.experimental.pallas import tpu_sc as plsc

LANES = 16
EPS = 1e-6


def _rsqrt_scalar(v):
    i = lax.bitcast_convert_type(v, jnp.int32)
    i = jnp.int32(0x5F3759DF) - lax.shift_right_logical(i, 1)
    y = lax.bitcast_convert_type(i, jnp.float32)
    for _ in range(3):
        y = y * (1.5 - 0.5 * v * y * y)
    return y


@functools.lru_cache(maxsize=None)
def _build(ntok, hidden, tvocab, chunk):
    info = plsc.get_sparse_core_info()
    nw = info.num_cores * info.num_subcores
    assert ntok % (nw * chunk) == 0
    tpw = ntok // nw
    nchunks = tpw // chunk
    dchunks = hidden // LANES
    inv_h = 1.0 / hidden
    mesh = plsc.VectorSubcoreMesh(core_axis_name="c", subcore_axis_name="s")

    @functools.partial(
        pl.kernel,
        out_type=jax.ShapeDtypeStruct((ntok, hidden), jnp.float32),
        mesh=mesh,
        scratch_types=[
            pltpu.VMEM((2, chunk), jnp.int32),
            pltpu.VMEM((2, chunk), jnp.int32),
            pltpu.VMEM((2, chunk + LANES), jnp.int32),
            pltpu.VMEM((chunk, hidden), jnp.float32),
            pltpu.VMEM((chunk, hidden), jnp.float32),
            pltpu.VMEM((chunk, hidden), jnp.float32),
            pltpu.VMEM((chunk, hidden), jnp.float32),
            pltpu.VMEM((tvocab, hidden), jnp.float32),
            pltpu.VMEM((hidden,), jnp.float32),
            pltpu.VMEM((hidden,), jnp.float32),
            pltpu.VMEM((2 * LANES,), jnp.float32),
            pltpu.SemaphoreType.DMA,
            pltpu.SemaphoreType.DMA,
            pltpu.SemaphoreType.DMA,
            pltpu.SemaphoreType.DMA,
            pltpu.SemaphoreType.DMA,
            pltpu.SemaphoreType.DMA,
        ],
    )
    def sc_kernel(ids_hbm, pids_hbm, tids_hbm, wtab_hbm, ptab_hbm, ttab_hbm,
                  lnw_hbm, lnb_hbm, out_hbm,
                  widx_v, pidx_v, tids_v, wbuf0, wbuf1, pbuf0, pbuf1,
                  ttab_v, lnw_v, lnb_v, red_v,
                  sem_w0, sem_w1, sem_p0, sem_p1, sem_o0, sem_o1):
        wid = lax.axis_index("s") * info.num_cores + lax.axis_index("c")
        base = wid * tpw
        wbufs = (wbuf0, wbuf1)
        pbufs = (pbuf0, pbuf1)
        wsems = (sem_w0, sem_w1)
        psems = (sem_p0, sem_p1)
        osems = (sem_o0, sem_o1)

        pltpu.sync_copy(ttab_hbm, ttab_v)
        pltpu.sync_copy(lnw_hbm, lnw_v)
        pltpu.sync_copy(lnb_hbm, lnb_v)
        red_v[pl.ds(LANES, LANES)] = jnp.zeros((LANES,), jnp.float32)

        def lane_sum(v):
            for sh in (8, 4, 2, 1):
                red_v[pl.ds(0, LANES)] = v
                v = v + red_v[pl.ds(sh, LANES)]
            return v[0]

        def idx_load(c, p):
            sl = pl.ds(base + c * chunk, chunk)
            pltpu.sync_copy(ids_hbm.at[sl], widx_v.at[p])
            pltpu.sync_copy(pids_hbm.at[sl], pidx_v.at[p])
            pltpu.sync_copy(tids_hbm.at[sl], tids_v.at[p, pl.ds(0, chunk)])

        def start_gathers(p):
            gw = pltpu.async_copy(
                wtab_hbm.at[widx_v.at[p]], wbufs[p], wsems[p])
            gp = pltpu.async_copy(
                ptab_hbm.at[pidx_v.at[p]], pbufs[p], psems[p])
            return gw, gp

        def start_out(c, p):
            return pltpu.async_copy(
                wbufs[p], out_hbm.at[pl.ds(base + c * chunk, chunk)],
                osems[p])

        def compute_chunk(p):
            wbuf = wbufs[p]
            pbuf = pbufs[p]
            tids = tids_v.at[p]

            def tok_body(i, tcarry):
                t = tids[pl.ds(i, LANES)][0]

                def acc_body(d, accs):
                    a1, a2 = accs
                    sl = pl.ds(d * LANES, LANES)
                    x = wbuf[i, sl] + pbuf[i, sl] + ttab_v[t, sl]
                    wbuf[i, sl] = x
                    return a1 + x, a2 + x * x
                acc1, acc2 = lax.fori_loop(
                    0, dchunks, acc_body,
                    (jnp.zeros((LANES,), jnp.float32),
                     jnp.zeros((LANES,), jnp.float32)),
                    unroll=12)
                mean = lane_sum(acc1) * inv_h
                var = lane_sum(acc2) * inv_h - mean * mean
                r = _rsqrt_scalar(var + EPS)

                def norm_body(d, ncarry):
                    sl = pl.ds(d * LANES, LANES)
                    y = (wbuf[i, sl] - mean) * (r * lnw_v[sl]) + lnb_v[sl]
                    wbuf[i, sl] = y
                    return ncarry
                lax.fori_loop(0, dchunks, norm_body, 0, unroll=12)
                return tcarry
            lax.fori_loop(0, chunk, tok_body, 0)

        idx_load(0, 0)
        gw, gp = start_gathers(0)
        gw.wait()
        gp.wait()

        out_dmas = [None, None]
        for c in range(nchunks):
            p = c % 2
            q = 1 - p
            if c + 1 < nchunks:
                if out_dmas[q] is not None:
                    out_dmas[q].wait()
                    out_dmas[q] = None
                idx_load(c + 1, q)
                gw, gp = start_gathers(q)
            compute_chunk(p)
            out_dmas[p] = start_out(c, p)
            if c + 1 < nchunks:
                gw.wait()
                gp.wait()
        for dma in out_dmas:
            if dma is not None:
                dma.wait()

    return sc_kernel


def kernel(input_ids, token_type_ids, position_ids, attention_mask,
           word_emb, pos_emb, type_emb, ln_weight, ln_bias):
    b, s = input_ids.shape
    ntok = b * s
    hidden = word_emb.shape[1]
    ids = input_ids.reshape(ntok).astype(jnp.int32)
    pids = position_ids.reshape(ntok).astype(jnp.int32)
    tids = token_type_ids.reshape(ntok).astype(jnp.int32)
    fn = _build(ntok, hidden, type_emb.shape[0], 32)
    out = fn(ids, pids, tids, word_emb, pos_emb, type_emb,
             ln_weight, ln_bias)
    return out.reshape(b, s, hidden)


# speedup vs baseline: 1.5803x; 1.5032x over previous
import functools

import jax
import jax.numpy as jnp
from jax import lax
from jax.experimental import pallas as pl
from jax.experimental.pallas import tpu as pltpu
from jax.experimental.pallas import tpu_sc as plsc

LANES = 16
EPS = 1e-6


def _rsqrt_scalar(v):
    i = lax.bitcast_convert_type(v, jnp.int32)
    i = jnp.int32(0x5F3759DF) - lax.shift_right_logical(i, 1)
    y = lax.bitcast_convert_type(i, jnp.float32)
    for _ in range(2):
        y = y * (1.5 - 0.5 * v * y * y)
    return y


@functools.lru_cache(maxsize=None)
def _build(ntok, hidden, tvocab, chunk):
    info = plsc.get_sparse_core_info()
    nw = info.num_cores * info.num_subcores
    assert ntok % (nw * chunk) == 0
    tpw = ntok // nw
    nchunks = tpw // chunk
    dchunks = hidden // LANES
    inv_h = 1.0 / hidden
    mesh = plsc.VectorSubcoreMesh(core_axis_name="c", subcore_axis_name="s")

    @functools.partial(
        pl.kernel,
        out_type=jax.ShapeDtypeStruct((ntok, hidden), jnp.float32),
        mesh=mesh,
        scratch_types=[
            pltpu.VMEM((2, chunk), jnp.int32),
            pltpu.VMEM((2, chunk), jnp.int32),
            pltpu.VMEM((2, chunk + LANES), jnp.int32),
            pltpu.VMEM((chunk, hidden), jnp.float32),
            pltpu.VMEM((chunk, hidden), jnp.float32),
            pltpu.VMEM((chunk, hidden), jnp.float32),
            pltpu.VMEM((chunk, hidden), jnp.float32),
            pltpu.VMEM((tvocab, hidden), jnp.float32),
            pltpu.VMEM((hidden,), jnp.float32),
            pltpu.VMEM((hidden,), jnp.float32),
            pltpu.VMEM((8 * LANES,), jnp.float32),
            pltpu.SemaphoreType.DMA,
            pltpu.SemaphoreType.DMA,
            pltpu.SemaphoreType.DMA,
            pltpu.SemaphoreType.DMA,
            pltpu.SemaphoreType.DMA,
            pltpu.SemaphoreType.DMA,
        ],
    )
    def sc_kernel(ids_hbm, pids_hbm, tids_hbm, wtab_hbm, ptab_hbm, ttab_hbm,
                  lnw_hbm, lnb_hbm, out_hbm,
                  widx_v, pidx_v, tids_v, wbuf0, wbuf1, pbuf0, pbuf1,
                  ttab_v, lnw_v, lnb_v, red_v,
                  sem_w0, sem_w1, sem_p0, sem_p1, sem_o0, sem_o1):
        wid = lax.axis_index("s") * info.num_cores + lax.axis_index("c")
        base = wid * tpw
        wbufs = (wbuf0, wbuf1)
        pbufs = (pbuf0, pbuf1)
        wsems = (sem_w0, sem_w1)
        psems = (sem_p0, sem_p1)
        osems = (sem_o0, sem_o1)

        pltpu.sync_copy(ttab_hbm, ttab_v)
        pltpu.sync_copy(lnw_hbm, lnw_v)
        pltpu.sync_copy(lnb_hbm, lnb_v)
        zeros = jnp.zeros((LANES,), jnp.float32)
        for z in range(4):
            red_v[pl.ds(2 * z * LANES + LANES, LANES)] = zeros

        def lane_sum4(vs):
            for sh in (8, 4, 2, 1):
                for z in range(4):
                    red_v[pl.ds(2 * z * LANES, LANES)] = vs[z]
                vs = [vs[z] + red_v[pl.ds(2 * z * LANES + sh, LANES)]
                      for z in range(4)]
            return [v[0] for v in vs]

        def idx_load(c, p):
            sl = pl.ds(base + c * chunk, chunk)
            pltpu.sync_copy(ids_hbm.at[sl], widx_v.at[p])
            pltpu.sync_copy(pids_hbm.at[sl], pidx_v.at[p])
            pltpu.sync_copy(tids_hbm.at[sl], tids_v.at[p, pl.ds(0, chunk)])

        def start_gathers(p):
            gw = pltpu.async_copy(
                wtab_hbm.at[widx_v.at[p]], wbufs[p], wsems[p])
            gp = pltpu.async_copy(
                ptab_hbm.at[pidx_v.at[p]], pbufs[p], psems[p])
            return gw, gp

        def start_out(c, p):
            return pltpu.async_copy(
                wbufs[p], out_hbm.at[pl.ds(base + c * chunk, chunk)],
                osems[p])

        def compute_chunk(p):
            wbuf = wbufs[p]
            pbuf = pbufs[p]
            tids = tids_v.at[p]

            def pair_body(k, tcarry):
                ia = 2 * k
                ib = ia + 1
                tvec = tids[pl.ds(ia, LANES)]
                ta = tvec[0]
                tb = tvec[1]
                def acc_body(d, accs):
                    a1, a2, b1, b2 = accs
                    sl = pl.ds(d * LANES, LANES)
                    xa = wbuf[ia, sl] + pbuf[ia, sl] + ttab_v[ta, sl]
                    xb = wbuf[ib, sl] + pbuf[ib, sl] + ttab_v[tb, sl]
                    wbuf[ia, sl] = xa
                    wbuf[ib, sl] = xb
                    return (a1 + xa, a2 + xa * xa, b1 + xb, b2 + xb * xb)
                z = jnp.zeros((LANES,), jnp.float32)
                a1, a2, b1, b2 = lax.fori_loop(
                    0, dchunks, acc_body, (z, z, z, z), unroll=12)
                s1a, s2a, s1b, s2b = lane_sum4([a1, a2, b1, b2])
                mean_a = s1a * inv_h
                mean_b = s1b * inv_h
                var_a = s2a * inv_h - mean_a * mean_a
                var_b = s2b * inv_h - mean_b * mean_b
                ra = _rsqrt_scalar(var_a + EPS)
                rb = _rsqrt_scalar(var_b + EPS)
                def norm_body(d, ncarry):
                    sl = pl.ds(d * LANES, LANES)
                    w = lnw_v[sl]
                    bi = lnb_v[sl]
                    ya = (wbuf[ia, sl] - mean_a) * (ra * w) + bi
                    yb = (wbuf[ib, sl] - mean_b) * (rb * w) + bi
                    wbuf[ia, sl] = ya
                    wbuf[ib, sl] = yb
                    return ncarry
                lax.fori_loop(0, dchunks, norm_body, 0, unroll=12)
                return tcarry
            lax.fori_loop(0, chunk // 2, pair_body, 0)

        idx_load(0, 0)
        gw, gp = start_gathers(0)
        gw.wait()
        gp.wait()

        out_dmas = [None, None]
        for c in range(nchunks):
            p = c % 2
            q = 1 - p
            if c + 1 < nchunks:
                if out_dmas[q] is not None:
                    out_dmas[q].wait()
                    out_dmas[q] = None
                idx_load(c + 1, q)
                gw, gp = start_gathers(q)
            compute_chunk(p)
            out_dmas[p] = start_out(c, p)
            if c + 1 < nchunks:
                gw.wait()
                gp.wait()
        for dma in out_dmas:
            if dma is not None:
                dma.wait()

    return sc_kernel


def kernel(input_ids, token_type_ids, position_ids, attention_mask,
           word_emb, pos_emb, type_emb, ln_weight, ln_bias):
    b, s = input_ids.shape
    ntok = b * s
    hidden = word_emb.shape[1]
    ids = input_ids.reshape(ntok).astype(jnp.int32)
    pids = position_ids.reshape(ntok).astype(jnp.int32)
    tids = token_type_ids.reshape(ntok).astype(jnp.int32)
    fn = _build(ntok, hidden, type_emb.shape[0], 32)
    out = fn(ids, pids, tids, word_emb, pos_emb, type_emb,
             ln_weight, ln_bias)
    return out.reshape(b, s, hidden)
